# baseline (speedup 1.0000x reference)
import functools

import jax
import jax.numpy as jnp
from jax import lax
from jax.experimental import pallas as pl
from jax.experimental.pallas import tpu as pltpu

N = 10000
E = 160000
D = 256
H = 256
C = 16
ROWS_BLK = 1000


def _tc1_body(x_ref, wp_ref, bp_ref, ws_ref, hp_ref, hs_ref):
    x = x_ref[...]
    dn = (((1,), (1,)), ((), ()))
    hp = lax.dot_general(x, wp_ref[...], dn, preferred_element_type=jnp.float32)
    hp = jnp.maximum(hp + bp_ref[...], 0.0)
    hs = lax.dot_general(x, ws_ref[...], dn, preferred_element_type=jnp.float32)
    hp_ref[...] = hp
    hs_ref[...] = hs


def _tc2_body(hs_ref, ng_ref, wn_ref, cb_ref, wl_ref, bl_ref, out_ref):
    dn = (((1,), (1,)), ((), ()))
    h = hs_ref[...] + lax.dot_general(
        ng_ref[...], wn_ref[...], dn, preferred_element_type=jnp.float32
    ) + cb_ref[...]
    h = jnp.where(h >= 0.0, h, 0.01 * h)
    o = lax.dot_general(h, wl_ref[...], dn, preferred_element_type=jnp.float32)
    out_ref[...] = jax.nn.sigmoid(o + bl_ref[...])


def _tc1(x, wp, bp, ws):
    grid = (N // ROWS_BLK,)
    return pl.pallas_call(
        _tc1_body,
        grid=grid,
        in_specs=[
            pl.BlockSpec((ROWS_BLK, D), lambda i: (i, 0)),
            pl.BlockSpec((D, D), lambda i: (0, 0)),
            pl.BlockSpec((1, D), lambda i: (0, 0)),
            pl.BlockSpec((H, D), lambda i: (0, 0)),
        ],
        out_specs=[
            pl.BlockSpec((ROWS_BLK, D), lambda i: (i, 0)),
            pl.BlockSpec((ROWS_BLK, H), lambda i: (i, 0)),
        ],
        out_shape=[
            jax.ShapeDtypeStruct((N, D), jnp.float32),
            jax.ShapeDtypeStruct((N, H), jnp.float32),
        ],
    )(x, wp, bp.reshape(1, D), ws)


def _tc2(h_self, neigh, wn, cb, wl, bl):
    grid = (N // ROWS_BLK,)
    return pl.pallas_call(
        _tc2_body,
        grid=grid,
        in_specs=[
            pl.BlockSpec((ROWS_BLK, H), lambda i: (i, 0)),
            pl.BlockSpec((ROWS_BLK, D), lambda i: (i, 0)),
            pl.BlockSpec((H, D), lambda i: (0, 0)),
            pl.BlockSpec((1, H), lambda i: (0, 0)),
            pl.BlockSpec((C, H), lambda i: (0, 0)),
            pl.BlockSpec((1, C), lambda i: (0, 0)),
        ],
        out_specs=pl.BlockSpec((ROWS_BLK, C), lambda i: (i, 0)),
        out_shape=jax.ShapeDtypeStruct((N, C), jnp.float32),
    )(h_self, neigh, wn, cb.reshape(1, H), wl, bl.reshape(1, C))


def _segment_max_sc(h_pool, edge_index):
    src = edge_index[0]
    dst = edge_index[1]
    msgs = jnp.take(h_pool, src, axis=0)
    neigh = jax.ops.segment_max(msgs, dst, num_segments=N)
    return jnp.maximum(neigh, 0.0)


def kernel(in_feat, edge_index, W_pool, b_pool, W_self, W_neigh, conv_bias, W_lin, b_lin):
    h_pool, h_self = _tc1(in_feat, W_pool, b_pool, W_self)
    neigh = _segment_max_sc(h_pool, edge_index)
    return _tc2(h_self, neigh, W_neigh, conv_bias, W_lin, b_lin)


# speedup vs baseline: 1.2518x; 1.2518x over previous
import dataclasses
import functools

import jax
import jax.numpy as jnp
from jax import lax
from jax.experimental import pallas as pl
from jax.experimental.pallas import tpu as pltpu
from jax.experimental.pallas import tpu_sc as plsc

N = 10000
E = 160000
D = 256
H = 256
C = 16
ROWS_BLK = 1000

NC = 2
NS = 16
NW = NC * NS
L = 16
NPW = 320
NOUT = NW * NPW
CH = 3200
NCHUNKS = E // CH
MBUF = 1024
FLUSH_AT = MBUF - L
GCH = 32
FV = D // L


def _tc1_body(x_ref, wp_ref, bp_ref, ws_ref, hp_ref, hs_ref):
    x = x_ref[...]
    dn = (((1,), (1,)), ((), ()))
    hp = lax.dot_general(x, wp_ref[...], dn, preferred_element_type=jnp.float32)
    hp = jnp.maximum(hp + bp_ref[...], 0.0)
    hs = lax.dot_general(x, ws_ref[...], dn, preferred_element_type=jnp.float32)
    hp_ref[...] = hp
    hs_ref[...] = hs


def _tc2_body(hs_ref, ng_ref, wn_ref, cb_ref, wl_ref, bl_ref, out_ref):
    dn = (((1,), (1,)), ((), ()))
    h = hs_ref[...] + lax.dot_general(
        ng_ref[...], wn_ref[...], dn, preferred_element_type=jnp.float32
    ) + cb_ref[...]
    h = jnp.where(h >= 0.0, h, 0.01 * h)
    o = lax.dot_general(h, wl_ref[...], dn, preferred_element_type=jnp.float32)
    out_ref[...] = jax.nn.sigmoid(o + bl_ref[...])


def _tc1(x, wp, bp, ws):
    grid = (N // ROWS_BLK,)
    return pl.pallas_call(
        _tc1_body,
        grid=grid,
        in_specs=[
            pl.BlockSpec((ROWS_BLK, D), lambda i: (i, 0)),
            pl.BlockSpec((D, D), lambda i: (0, 0)),
            pl.BlockSpec((1, D), lambda i: (0, 0)),
            pl.BlockSpec((H, D), lambda i: (0, 0)),
        ],
        out_specs=[
            pl.BlockSpec((ROWS_BLK, D), lambda i: (i, 0)),
            pl.BlockSpec((ROWS_BLK, H), lambda i: (i, 0)),
        ],
        out_shape=[
            jax.ShapeDtypeStruct((N, D), jnp.float32),
            jax.ShapeDtypeStruct((N, H), jnp.float32),
        ],
    )(x, wp, bp.reshape(1, D), ws)


def _tc2(h_self, neigh, wn, cb, wl, bl):
    grid = (N // ROWS_BLK,)
    return pl.pallas_call(
        _tc2_body,
        grid=grid,
        in_specs=[
            pl.BlockSpec((ROWS_BLK, H), lambda i: (i, 0)),
            pl.BlockSpec((ROWS_BLK, D), lambda i: (i, 0)),
            pl.BlockSpec((H, D), lambda i: (0, 0)),
            pl.BlockSpec((1, H), lambda i: (0, 0)),
            pl.BlockSpec((C, H), lambda i: (0, 0)),
            pl.BlockSpec((1, C), lambda i: (0, 0)),
        ],
        out_specs=pl.BlockSpec((ROWS_BLK, C), lambda i: (i, 0)),
        out_shape=jax.ShapeDtypeStruct((N, C), jnp.float32),
    )(h_self, neigh, wn, cb.reshape(1, H), wl, bl.reshape(1, C))


def _sc_body(hpool, ei, out, acc, eb, msrc, mdstl, grows, sem0, sem1, semg):
    cidx = lax.axis_index("c")
    sidx = lax.axis_index("s")
    wid = sidx * NC + cidx
    lo = wid * NPW
    hi = lo + NPW
    iota = lax.iota(jnp.int32, L)
    zv = jnp.zeros((L,), jnp.float32)

    def zrow(r, carry):
        for f in range(FV):
            acc[r, pl.ds(f * L, L)] = zv
        return carry

    lax.fori_loop(0, NPW + 1, zrow, 0)

    def flush(off):
        off_pad = ((off + GCH - 1) // GCH) * GCH
        for j in range(GCH // L):
            idx = off + j * L + iota
            mpad = idx < off_pad
            idxc = jnp.where(mpad, idx, 0)
            plsc.store_scatter(msrc, [idxc], jnp.zeros((L,), jnp.int32), mask=mpad)
            plsc.store_scatter(mdstl, [idxc], jnp.full((L,), NPW, jnp.int32), mask=mpad)
        nch = off_pad // GCH

        def gbody(g, carry):
            base = g * GCH
            pltpu.async_copy(
                hpool.at[msrc.at[pl.ds(base, GCH)]], grows, semg
            ).wait()
            for sub in range(GCH // L):
                dv16 = mdstl[pl.ds(base + sub * L, L)]
                for e in range(L):
                    d = dv16[e]
                    row = sub * L + e
                    for f in range(FV):
                        sl = pl.ds(f * L, L)
                        acc[d, sl] = jnp.maximum(acc[d, sl], grows[row, sl])
            return carry

        lax.fori_loop(0, nch, gbody, 0)
        return jnp.int32(0)

    def scan_chunk(b, off):
        def vbody(v, off):
            sv = eb[b, 0, pl.ds(v * L, L)]
            dv = eb[b, 1, pl.ds(v * L, L)]
            m = (dv >= lo) & (dv < hi)
            mi = jnp.where(m, 1, 0).astype(jnp.int32)
            c = plsc.cumsum(mi)
            pos = jnp.where(m, off + c - 1, 0)
            plsc.store_scatter(msrc, [pos], sv, mask=m)
            plsc.store_scatter(mdstl, [pos], dv - lo, mask=m)
            off = off + jnp.sum(mi)
            return lax.cond(off >= FLUSH_AT, flush, lambda o: o, off)

        return lax.fori_loop(0, CH // L, vbody, off)

    pltpu.async_copy(ei.at[:, pl.ds(0, CH)], eb.at[0], sem0)

    def pair(c2, off):
        for b in range(2):
            c = 2 * c2 + b
            sem_cur = sem0 if b == 0 else sem1
            sem_nxt = sem1 if b == 0 else sem0

            @pl.when(c + 1 < NCHUNKS)
            def _():
                pltpu.async_copy(
                    ei.at[:, pl.ds((c + 1) * CH, CH)], eb.at[1 - b], sem_nxt
                )

            pltpu.make_async_copy(
                ei.at[:, pl.ds(c * CH, CH)], eb.at[b], sem_cur
            ).wait()
            off = scan_chunk(b, off)
        return off

    off = lax.fori_loop(0, NCHUNKS // 2, pair, jnp.int32(0))
    flush(off)
    pltpu.sync_copy(acc.at[pl.ds(0, NPW)], out.at[pl.ds(lo, NPW)])


def _segment_max_sc(h_pool, edge_index):
    mesh = plsc.VectorSubcoreMesh(core_axis_name="c", subcore_axis_name="s")
    cp = pltpu.CompilerParams()
    if "needs_layout_passes" in pltpu.CompilerParams.__dataclass_fields__:
        cp = dataclasses.replace(cp, needs_layout_passes=False)
    f = pl.kernel(
        _sc_body,
        out_type=jax.ShapeDtypeStruct((NOUT, D), jnp.float32),
        mesh=mesh,
        compiler_params=cp,
        scratch_types=[
            pltpu.VMEM((NPW + 1, D), jnp.float32),
            pltpu.VMEM((2, 2, CH), jnp.int32),
            pltpu.VMEM((MBUF,), jnp.int32),
            pltpu.VMEM((MBUF,), jnp.int32),
            pltpu.VMEM((GCH, D), jnp.float32),
            pltpu.SemaphoreType.DMA,
            pltpu.SemaphoreType.DMA,
            pltpu.SemaphoreType.DMA,
        ],
    )
    return f(h_pool, edge_index)[:N]


def kernel(in_feat, edge_index, W_pool, b_pool, W_self, W_neigh, conv_bias, W_lin, b_lin):
    h_pool, h_self = _tc1(in_feat, W_pool, b_pool, W_self)
    neigh = _segment_max_sc(h_pool, edge_index)
    return _tc2(h_self, neigh, W_neigh, conv_bias, W_lin, b_lin)


# speedup vs baseline: 1.4215x; 1.1356x over previous
import dataclasses
import functools

import jax
import jax.numpy as jnp
from jax import lax
from jax.experimental import pallas as pl
from jax.experimental.pallas import tpu as pltpu
from jax.experimental.pallas import tpu_sc as plsc

N = 10000
E = 160000
D = 256
H = 256
C = 16
ROWS_BLK = 1000

NC = 2
NS = 16
NW = NC * NS
L = 16
NPW = 320
NOUT = NW * NPW
CH = 3200
NCHUNKS = E // CH
MBUF = 1024
FLUSH_AT = MBUF - L
GCH = 16
FV = D // L


def _tc1_body(x_ref, wp_ref, bp_ref, ws_ref, hp_ref, hs_ref):
    x = x_ref[...]
    dn = (((1,), (1,)), ((), ()))
    hp = lax.dot_general(x, wp_ref[...], dn, preferred_element_type=jnp.float32)
    hp = jnp.maximum(hp + bp_ref[...], 0.0)
    hs = lax.dot_general(x, ws_ref[...], dn, preferred_element_type=jnp.float32)
    hp_ref[...] = hp
    hs_ref[...] = hs


def _tc2_body(hs_ref, ng_ref, wn_ref, cb_ref, wl_ref, bl_ref, out_ref):
    dn = (((1,), (1,)), ((), ()))
    h = hs_ref[...] + lax.dot_general(
        ng_ref[...], wn_ref[...], dn, preferred_element_type=jnp.float32
    ) + cb_ref[...]
    h = jnp.where(h >= 0.0, h, 0.01 * h)
    o = lax.dot_general(h, wl_ref[...], dn, preferred_element_type=jnp.float32)
    out_ref[...] = jax.nn.sigmoid(o + bl_ref[...])


def _tc1(x, wp, bp, ws):
    grid = (N // ROWS_BLK,)
    return pl.pallas_call(
        _tc1_body,
        grid=grid,
        in_specs=[
            pl.BlockSpec((ROWS_BLK, D), lambda i: (i, 0)),
            pl.BlockSpec((D, D), lambda i: (0, 0)),
            pl.BlockSpec((1, D), lambda i: (0, 0)),
            pl.BlockSpec((H, D), lambda i: (0, 0)),
        ],
        out_specs=[
            pl.BlockSpec((ROWS_BLK, D), lambda i: (i, 0)),
            pl.BlockSpec((ROWS_BLK, H), lambda i: (i, 0)),
        ],
        out_shape=[
            jax.ShapeDtypeStruct((N, D), jnp.float32),
            jax.ShapeDtypeStruct((N, H), jnp.float32),
        ],
    )(x, wp, bp.reshape(1, D), ws)


def _tc2(h_self, neigh, wn, cb, wl, bl):
    grid = (N // ROWS_BLK,)
    return pl.pallas_call(
        _tc2_body,
        grid=grid,
        in_specs=[
            pl.BlockSpec((ROWS_BLK, H), lambda i: (i, 0)),
            pl.BlockSpec((ROWS_BLK, D), lambda i: (i, 0)),
            pl.BlockSpec((H, D), lambda i: (0, 0)),
            pl.BlockSpec((1, H), lambda i: (0, 0)),
            pl.BlockSpec((C, H), lambda i: (0, 0)),
            pl.BlockSpec((1, C), lambda i: (0, 0)),
        ],
        out_specs=pl.BlockSpec((ROWS_BLK, C), lambda i: (i, 0)),
        out_shape=jax.ShapeDtypeStruct((N, C), jnp.float32),
    )(h_self, neigh, wn, cb.reshape(1, H), wl, bl.reshape(1, C))


def _sc_body(hpool, ei, out, acc, eb, msrc, mdstl, grows, sem0, sem1, semg):
    cidx = lax.axis_index("c")
    sidx = lax.axis_index("s")
    wid = sidx * NC + cidx
    lo = wid * NPW
    hi = lo + NPW
    iota = lax.iota(jnp.int32, L)
    zv = jnp.zeros((L,), jnp.float32)

    def zrow(r, carry):
        for f in range(FV):
            acc[r, pl.ds(f * L, L)] = zv
        return carry

    lax.fori_loop(0, NPW + 1, zrow, 0)

    def mfill(j, carry):
        msrc[pl.ds(j * L, L)] = jnp.zeros((L,), jnp.int32)
        mdstl[pl.ds(j * L, L)] = jnp.full((L,), NPW, jnp.int32)
        return carry

    lax.fori_loop(0, MBUF // L, mfill, 0)

    def _accum_chunk(g, b):
        base = g * GCH
        for sub in range(GCH // L):
            dv16 = mdstl[pl.ds(base + sub * L, L)]
            for e in range(L):
                d = dv16[e]
                row = sub * L + e
                for f in range(FV):
                    sl = pl.ds(f * L, L)
                    acc[d, sl] = jnp.maximum(acc[d, sl], grows[b, row, sl])

    def _fire(g, b):
        pltpu.async_copy(
            hpool.at[msrc.at[pl.ds(g * GCH, GCH)]], grows.at[b], semg
        )

    def _gwait(g, b):
        pltpu.make_async_copy(
            hpool.at[msrc.at[pl.ds(g * GCH, GCH)]], grows.at[b], semg
        ).wait()

    def flush_full(off):
        nch = MBUF // GCH
        _fire(0, 0)

        def gpair(g2, carry):
            for b in range(2):
                g = 2 * g2 + b

                @pl.when(g + 1 < nch)
                def _():
                    _fire(g + 1, 1 - b)

                _gwait(g, b)
                _accum_chunk(g, b)
            return carry

        lax.fori_loop(0, nch // 2, gpair, 0)
        return jnp.int32(0)

    def scan_chunk(b, off):
        def vbody(v, off):
            sv = eb[b, 0, pl.ds(v * L, L)]
            dv = eb[b, 1, pl.ds(v * L, L)]
            m = (dv >= lo) & (dv < hi)
            mi = jnp.where(m, 1, 0).astype(jnp.int32)
            c = plsc.cumsum(mi)
            pos = jnp.where(m, off + c - 1, 0)
            plsc.store_scatter(msrc, [pos], sv, mask=m)
            plsc.store_scatter(mdstl, [pos], dv - lo, mask=m)
            off = off + jnp.sum(mi)
            return lax.cond(off >= FLUSH_AT, flush_full, lambda o: o, off)

        return lax.fori_loop(0, CH // L, vbody, off)

    pltpu.async_copy(ei.at[:, pl.ds(0, CH)], eb.at[0], sem0)

    def pair(c2, off):
        for b in range(2):
            c = 2 * c2 + b
            sem_cur = sem0 if b == 0 else sem1
            sem_nxt = sem1 if b == 0 else sem0

            @pl.when(c + 1 < NCHUNKS)
            def _():
                pltpu.async_copy(
                    ei.at[:, pl.ds((c + 1) * CH, CH)], eb.at[1 - b], sem_nxt
                )

            pltpu.make_async_copy(
                ei.at[:, pl.ds(c * CH, CH)], eb.at[b], sem_cur
            ).wait()
            off = scan_chunk(b, off)
        return off

    off = lax.fori_loop(0, NCHUNKS // 2, pair, jnp.int32(0))
    flush_full(off)
    pltpu.sync_copy(acc.at[pl.ds(0, NPW)], out.at[pl.ds(lo, NPW)])


def _segment_max_sc(h_pool, edge_index):
    mesh = plsc.VectorSubcoreMesh(core_axis_name="c", subcore_axis_name="s")
    cp = pltpu.CompilerParams()
    if "needs_layout_passes" in pltpu.CompilerParams.__dataclass_fields__:
        cp = dataclasses.replace(cp, needs_layout_passes=False)
    f = pl.kernel(
        _sc_body,
        out_type=jax.ShapeDtypeStruct((NOUT, D), jnp.float32),
        mesh=mesh,
        compiler_params=cp,
        scratch_types=[
            pltpu.VMEM((NPW + 1, D), jnp.float32),
            pltpu.VMEM((2, 2, CH), jnp.int32),
            pltpu.VMEM((MBUF,), jnp.int32),
            pltpu.VMEM((MBUF,), jnp.int32),
            pltpu.VMEM((2, GCH, D), jnp.float32),
            pltpu.SemaphoreType.DMA,
            pltpu.SemaphoreType.DMA,
            pltpu.SemaphoreType.DMA,
        ],
    )
    return f(h_pool, edge_index)[:N]


def kernel(in_feat, edge_index, W_pool, b_pool, W_self, W_neigh, conv_bias, W_lin, b_lin):
    h_pool, h_self = _tc1(in_feat, W_pool, b_pool, W_self)
    neigh = _segment_max_sc(h_pool, edge_index)
    return _tc2(h_self, neigh, W_neigh, conv_bias, W_lin, b_lin)
